# initial kernel scaffold (unmeasured)
import jax
import jax.numpy as jnp
from jax import lax
from jax.experimental import pallas as pl
from jax.experimental.pallas import tpu as pltpu

N_DEV = 4
B, S, H, D = 2, 512, 8, 64
SCALE = D ** -0.5


def kernel(Q, K, V):
    def body(q_ref, k_ref, v_ref, out_ref,
             k_com, v_com, ksend, krecv, vsend, vrecv):
        mx = lax.axis_index("x")
        my = lax.axis_index("y")
        mz = lax.axis_index("z")
        left = (mz - 1) % N_DEV
        right = (mz + 1) % N_DEV

        barrier_sem = pltpu.get_barrier_semaphore()
        for nbr in (left, right):
            pl.semaphore_signal(
                barrier_sem, inc=1,
                device_id=(mx, my, nbr),
                device_id_type=pl.DeviceIdType.MESH,
            )
        pl.semaphore_wait(barrier_sem, 2)

        k_com[0] = jnp.transpose(k_ref[...], (0, 2, 1, 3)).astype(jnp.bfloat16)
        v_com[0] = jnp.transpose(v_ref[...], (0, 2, 1, 3)).astype(jnp.bfloat16)

        for h in range(N_DEV - 1):
            rk = pltpu.make_async_remote_copy(
                src_ref=k_com.at[h], dst_ref=k_com.at[h + 1],
                send_sem=ksend.at[h], recv_sem=krecv.at[h],
                device_id=(mx, my, right),
                device_id_type=pl.DeviceIdType.MESH,
            )
            rv = pltpu.make_async_remote_copy(
                src_ref=v_com.at[h], dst_ref=v_com.at[h + 1],
                send_sem=vsend.at[h], recv_sem=vrecv.at[h],
                device_id=(mx, my, right),
                device_id_type=pl.DeviceIdType.MESH,
            )
            rk.start()
            rv.start()
            rk.wait()
            rv.wait()

        for b in range(B):
            for hh in range(H):
                q = q_ref[b, :, hh, :].astype(jnp.bfloat16)
                kf = k_com[:, b, hh, :, :].reshape(N_DEV * S, D)
                vf = v_com[:, b, hh, :, :].reshape(N_DEV * S, D)
                s = lax.dot_general(
                    q, kf, (((1,), (1,)), ((), ())),
                    preferred_element_type=jnp.float32,
                ) * SCALE
                m = jnp.max(s, axis=1, keepdims=True)
                p = jnp.exp(s - m)
                l = jnp.sum(p, axis=1, keepdims=True)
                pv = lax.dot_general(
                    p.astype(jnp.bfloat16), vf, (((1,), (0,)), ((), ())),
                    preferred_element_type=jnp.float32,
                )
                out_ref[b, :, hh, :] = pv / l

    return pl.pallas_call(
        body,
        out_shape=jax.ShapeDtypeStruct((B, S, H, D), jnp.float32),
        in_specs=[pl.BlockSpec(memory_space=pltpu.VMEM)] * 3,
        out_specs=pl.BlockSpec(memory_space=pltpu.VMEM),
        scratch_shapes=[
            pltpu.VMEM((N_DEV, B, H, S, D), jnp.bfloat16),
            pltpu.VMEM((N_DEV, B, H, S, D), jnp.bfloat16),
            pltpu.SemaphoreType.DMA((N_DEV - 1,)),
            pltpu.SemaphoreType.DMA((N_DEV - 1,)),
            pltpu.SemaphoreType.DMA((N_DEV - 1,)),
            pltpu.SemaphoreType.DMA((N_DEV - 1,)),
        ],
        compiler_params=pltpu.CompilerParams(collective_id=0),
    )(Q, K, V)


# baseline (device time: 199839 ns/iter reference)
import jax
import jax.numpy as jnp
from jax import lax
from jax.experimental import pallas as pl
from jax.experimental.pallas import tpu as pltpu

N_DEV = 4
B, S, H, D = 2, 512, 8, 64
SCALE = D ** -0.5


def kernel(Q, K, V):
    def body(q_ref, k_ref, v_ref, out_ref,
             qt, k_com, v_com, ksend, krecv, vsend, vrecv):
        mx = lax.axis_index("x")
        my = lax.axis_index("y")
        mz = lax.axis_index("z")
        left = (mz - 1) % N_DEV
        right = (mz + 1) % N_DEV

        barrier_sem = pltpu.get_barrier_semaphore()
        for nbr in (left, right):
            pl.semaphore_signal(
                barrier_sem, inc=1,
                device_id=(mx, my, nbr),
                device_id_type=pl.DeviceIdType.MESH,
            )
        pl.semaphore_wait(barrier_sem, 2)

        k_com[0] = jnp.transpose(k_ref[...], (0, 2, 1, 3)).astype(jnp.bfloat16)
        v_com[0] = jnp.transpose(v_ref[...], (0, 2, 1, 3)).astype(jnp.bfloat16)
        qt[...] = jnp.transpose(q_ref[...], (0, 2, 1, 3)).astype(jnp.bfloat16)

        for h in range(N_DEV - 1):
            rk = pltpu.make_async_remote_copy(
                src_ref=k_com.at[h], dst_ref=k_com.at[h + 1],
                send_sem=ksend.at[h], recv_sem=krecv.at[h],
                device_id=(mx, my, right),
                device_id_type=pl.DeviceIdType.MESH,
            )
            rv = pltpu.make_async_remote_copy(
                src_ref=v_com.at[h], dst_ref=v_com.at[h + 1],
                send_sem=vsend.at[h], recv_sem=vrecv.at[h],
                device_id=(mx, my, right),
                device_id_type=pl.DeviceIdType.MESH,
            )
            rk.start()
            rv.start()
            rk.wait()
            rv.wait()

        def bh_step(bh, _):
            b = bh // H
            hh = bh % H
            q = qt[b, hh]
            m = jnp.full((S, 1), -1e30, jnp.float32)
            l = jnp.zeros((S, 1), jnp.float32)
            acc = jnp.zeros((S, D), jnp.float32)
            for slot in range(N_DEV):
                k = k_com[slot, b, hh]
                v = v_com[slot, b, hh]
                s = lax.dot_general(
                    q, k, (((1,), (1,)), ((), ())),
                    preferred_element_type=jnp.float32,
                ) * SCALE
                m_new = jnp.maximum(m, jnp.max(s, axis=1, keepdims=True))
                alpha = jnp.exp(m - m_new)
                p = jnp.exp(s - m_new)
                l = l * alpha + jnp.sum(p, axis=1, keepdims=True)
                acc = acc * alpha + lax.dot_general(
                    p.astype(jnp.bfloat16), v, (((1,), (0,)), ((), ())),
                    preferred_element_type=jnp.float32,
                )
                m = m_new
            out_ref[b, :, hh, :] = acc / l
            return 0

        lax.fori_loop(0, B * H, bh_step, 0)

    return pl.pallas_call(
        body,
        out_shape=jax.ShapeDtypeStruct((B, S, H, D), jnp.float32),
        in_specs=[pl.BlockSpec(memory_space=pltpu.VMEM)] * 3,
        out_specs=pl.BlockSpec(memory_space=pltpu.VMEM),
        scratch_shapes=[
            pltpu.VMEM((B, H, S, D), jnp.bfloat16),
            pltpu.VMEM((N_DEV, B, H, S, D), jnp.bfloat16),
            pltpu.VMEM((N_DEV, B, H, S, D), jnp.bfloat16),
            pltpu.SemaphoreType.DMA((N_DEV - 1,)),
            pltpu.SemaphoreType.DMA((N_DEV - 1,)),
            pltpu.SemaphoreType.DMA((N_DEV - 1,)),
            pltpu.SemaphoreType.DMA((N_DEV - 1,)),
        ],
        compiler_params=pltpu.CompilerParams(
            collective_id=0,
            vmem_limit_bytes=100 * 1024 * 1024,
        ),
    )(Q, K, V)


# device time: 165588 ns/iter; 1.2068x vs baseline; 1.2068x over previous
import jax
import jax.numpy as jnp
from jax import lax
from jax.experimental import pallas as pl
from jax.experimental.pallas import tpu as pltpu

N_DEV = 4
B, S, H, D = 2, 512, 8, 64
SCALE = D ** -0.5


def kernel(Q, K, V):
    def body(q_ref, k_ref, v_ref, out_ref,
             qt, k_com, v_com, ksend, krecv, vsend, vrecv):
        mx = lax.axis_index("x")
        my = lax.axis_index("y")
        mz = lax.axis_index("z")
        left = (mz - 1) % N_DEV
        right = (mz + 1) % N_DEV

        barrier_sem = pltpu.get_barrier_semaphore()
        for nbr in (left, right):
            pl.semaphore_signal(
                barrier_sem, inc=1,
                device_id=(mx, my, nbr),
                device_id_type=pl.DeviceIdType.MESH,
            )
        pl.semaphore_wait(barrier_sem, 2)

        k_com[0] = jnp.transpose(k_ref[...], (0, 2, 1, 3)).astype(jnp.bfloat16)
        v_com[0] = jnp.transpose(v_ref[...], (0, 2, 1, 3)).astype(jnp.bfloat16)
        qt[...] = jnp.transpose(q_ref[...], (0, 2, 1, 3)).astype(jnp.bfloat16)

        for h in range(N_DEV - 1):
            rk = pltpu.make_async_remote_copy(
                src_ref=k_com.at[h], dst_ref=k_com.at[h + 1],
                send_sem=ksend.at[h], recv_sem=krecv.at[h],
                device_id=(mx, my, right),
                device_id_type=pl.DeviceIdType.MESH,
            )
            rv = pltpu.make_async_remote_copy(
                src_ref=v_com.at[h], dst_ref=v_com.at[h + 1],
                send_sem=vsend.at[h], recv_sem=vrecv.at[h],
                device_id=(mx, my, right),
                device_id_type=pl.DeviceIdType.MESH,
            )
            rk.start()
            rv.start()
            rk.wait()
            rv.wait()

        def bh_step(bh, _):
            b = bh // H
            hh = bh % H
            q = qt[b, hh]
            m = jnp.full((S, 1), -1e30, jnp.float32)
            l = jnp.zeros((S, 1), jnp.float32)
            acc = jnp.zeros((S, D), jnp.float32)
            for slot in range(N_DEV):
                k = k_com[slot, b, hh]
                v = v_com[slot, b, hh]
                s = lax.dot_general(
                    q, k, (((1,), (1,)), ((), ())),
                    preferred_element_type=jnp.float32,
                ) * SCALE
                m_new = jnp.maximum(m, jnp.max(s, axis=1, keepdims=True))
                alpha = jnp.exp(m - m_new)
                p = jnp.exp(s - m_new)
                l = l * alpha + jnp.sum(p, axis=1, keepdims=True)
                acc = acc * alpha + lax.dot_general(
                    p.astype(jnp.bfloat16), v, (((1,), (0,)), ((), ())),
                    preferred_element_type=jnp.float32,
                )
                m = m_new
            out_ref[b, :, hh, :] = acc / l
            return 0

        import os as _os
        if _os.environ.get("ABLATE") != "comm_only":
            lax.fori_loop(0, B * H, bh_step, 0)
        else:
            out_ref[...] = q_ref[...]

    return pl.pallas_call(
        body,
        out_shape=jax.ShapeDtypeStruct((B, S, H, D), jnp.float32),
        in_specs=[pl.BlockSpec(memory_space=pltpu.VMEM)] * 3,
        out_specs=pl.BlockSpec(memory_space=pltpu.VMEM),
        scratch_shapes=[
            pltpu.VMEM((B, H, S, D), jnp.bfloat16),
            pltpu.VMEM((N_DEV, B, H, S, D), jnp.bfloat16),
            pltpu.VMEM((N_DEV, B, H, S, D), jnp.bfloat16),
            pltpu.SemaphoreType.DMA((N_DEV - 1,)),
            pltpu.SemaphoreType.DMA((N_DEV - 1,)),
            pltpu.SemaphoreType.DMA((N_DEV - 1,)),
            pltpu.SemaphoreType.DMA((N_DEV - 1,)),
        ],
        compiler_params=pltpu.CompilerParams(
            collective_id=0,
            vmem_limit_bytes=100 * 1024 * 1024,
        ),
    )(Q, K, V)


# device time: 127478 ns/iter; 1.5676x vs baseline; 1.2990x over previous
import jax
import jax.numpy as jnp
from jax import lax
from jax.experimental import pallas as pl
from jax.experimental.pallas import tpu as pltpu

N_DEV = 4
B, S, H, D = 2, 512, 8, 64
SCALE = D ** -0.5


def kernel(Q, K, V):
    def body(q_ref, k_ref, v_ref, out_ref,
             qt, kloc, vloc, k_com, v_com, acc, lsum,
             ksend, krecv, vsend, vrecv):
        mx = lax.axis_index("x")
        my = lax.axis_index("y")
        mz = lax.axis_index("z")

        barrier_sem = pltpu.get_barrier_semaphore()
        for j in range(N_DEV - 1):
            pl.semaphore_signal(
                barrier_sem, inc=1,
                device_id=(mx, my, (mz + 1 + j) % N_DEV),
                device_id_type=pl.DeviceIdType.MESH,
            )
        pl.semaphore_wait(barrier_sem, N_DEV - 1)

        def fill_step(bh, _):
            b = bh // H
            hh = bh % H
            qt[b, hh] = (q_ref[b, :, hh, :] * SCALE).astype(jnp.bfloat16)
            kloc[b, hh] = jnp.transpose(
                k_ref[b, :, hh, :], (1, 0)).astype(jnp.bfloat16)
            vloc[b, hh] = jnp.transpose(
                v_ref[b, :, hh, :], (1, 0)).astype(jnp.bfloat16)
            return 0

        lax.fori_loop(0, B * H, fill_step, 0)

        sends = []
        for j in range(N_DEV - 1):
            p = (mz + 1 + j) % N_DEV
            r = N_DEV - 2 - j
            rk = pltpu.make_async_remote_copy(
                src_ref=kloc, dst_ref=k_com.at[r],
                send_sem=ksend.at[j], recv_sem=krecv.at[r],
                device_id=(mx, my, p), device_id_type=pl.DeviceIdType.MESH,
            )
            rv = pltpu.make_async_remote_copy(
                src_ref=vloc, dst_ref=v_com.at[r],
                send_sem=vsend.at[j], recv_sem=vrecv.at[r],
                device_id=(mx, my, p), device_id_type=pl.DeviceIdType.MESH,
            )
            rk.start()
            rv.start()
            sends.append((rk, rv))

        def chunk_loop(k_at, v_at, first, last):
            def bh_step(bh, _):
                b = bh // H
                hh = bh % H
                q = qt[b, hh]
                kT = k_at(b, hh)
                vT = v_at(b, hh)
                s = lax.dot_general(
                    q, kT, (((1,), (0,)), ((), ())),
                    preferred_element_type=jnp.float32,
                )
                p = jnp.exp(s)
                pv = lax.dot_general(
                    p.astype(jnp.bfloat16), vT, (((1,), (1,)), ((), ())),
                    preferred_element_type=jnp.float32,
                )
                lv = jnp.sum(p, axis=1, keepdims=True)
                if first:
                    a = pv
                    l = lv
                else:
                    a = acc[b, hh] + pv
                    l = lsum[b, hh] + lv
                if last:
                    out_ref[b, :, hh, :] = a / l
                else:
                    acc[b, hh] = a
                    lsum[b, hh] = l
                return 0

            lax.fori_loop(0, B * H, bh_step, 0)

        chunk_loop(lambda b, hh: kloc[b, hh], lambda b, hh: vloc[b, hh],
                   first=True, last=False)

        for r in range(N_DEV - 1):
            wk = pltpu.make_async_remote_copy(
                src_ref=kloc, dst_ref=k_com.at[r],
                send_sem=ksend.at[0], recv_sem=krecv.at[r],
                device_id=(mx, my, mz), device_id_type=pl.DeviceIdType.MESH,
            )
            wv = pltpu.make_async_remote_copy(
                src_ref=vloc, dst_ref=v_com.at[r],
                send_sem=vsend.at[0], recv_sem=vrecv.at[r],
                device_id=(mx, my, mz), device_id_type=pl.DeviceIdType.MESH,
            )
            wk.wait_recv()
            wv.wait_recv()
            chunk_loop(lambda b, hh, r=r: k_com[r, b, hh],
                       lambda b, hh, r=r: v_com[r, b, hh],
                       first=False, last=(r == N_DEV - 2))

        for rk, rv in sends:
            rk.wait_send()
            rv.wait_send()

    return pl.pallas_call(
        body,
        out_shape=jax.ShapeDtypeStruct((B, S, H, D), jnp.float32),
        in_specs=[pl.BlockSpec(memory_space=pltpu.VMEM)] * 3,
        out_specs=pl.BlockSpec(memory_space=pltpu.VMEM),
        scratch_shapes=[
            pltpu.VMEM((B, H, S, D), jnp.bfloat16),
            pltpu.VMEM((B, H, D, S), jnp.bfloat16),
            pltpu.VMEM((B, H, D, S), jnp.bfloat16),
            pltpu.VMEM((N_DEV - 1, B, H, D, S), jnp.bfloat16),
            pltpu.VMEM((N_DEV - 1, B, H, D, S), jnp.bfloat16),
            pltpu.VMEM((B, H, S, D), jnp.float32),
            pltpu.VMEM((B, H, S, 1), jnp.float32),
            pltpu.SemaphoreType.DMA((N_DEV - 1,)),
            pltpu.SemaphoreType.DMA((N_DEV - 1,)),
            pltpu.SemaphoreType.DMA((N_DEV - 1,)),
            pltpu.SemaphoreType.DMA((N_DEV - 1,)),
        ],
        compiler_params=pltpu.CompilerParams(
            collective_id=0,
            vmem_limit_bytes=100 * 1024 * 1024,
        ),
    )(Q, K, V)


# device time: 73520 ns/iter; 2.7182x vs baseline; 1.7339x over previous
import jax
import jax.numpy as jnp
from jax import lax
from jax.experimental import pallas as pl
from jax.experimental.pallas import tpu as pltpu

N_DEV = 4
B, S, H, D = 2, 512, 8, 64
SCALE = D ** -0.5


import os
_ABLATE = os.environ.get("ABLATE", "")


def kernel(Q, K, V):
    def body(q_ref, k_ref, v_ref, out_ref,
             qt, kloc, vloc, k_com, v_com, acc, lsum,
             ksend, krecv, vsend, vrecv):
        mx = lax.axis_index("x")
        my = lax.axis_index("y")
        mz = lax.axis_index("z")

        barrier_sem = pltpu.get_barrier_semaphore()
        for j in range(N_DEV - 1):
            pl.semaphore_signal(
                barrier_sem, inc=1,
                device_id=(mx, my, (mz + 1 + j) % N_DEV),
                device_id_type=pl.DeviceIdType.MESH,
            )
        pl.semaphore_wait(barrier_sem, N_DEV - 1)

        def fill_step(bh, _):
            b = bh // H
            hh = bh % H
            qt[b, hh] = (q_ref[b, :, hh, :] * SCALE).astype(jnp.bfloat16)
            kloc[b, hh] = jnp.transpose(
                k_ref[b, :, hh, :], (1, 0)).astype(jnp.bfloat16)
            vloc[b, hh] = jnp.transpose(
                v_ref[b, :, hh, :], (1, 0)).astype(jnp.bfloat16)
            return 0

        lax.fori_loop(0, B * H, fill_step, 0)

        sends = []
        for j in range(N_DEV - 1 if _ABLATE != "compute" else 0):
            p = (mz + 1 + j) % N_DEV
            r = N_DEV - 2 - j
            rk = pltpu.make_async_remote_copy(
                src_ref=kloc, dst_ref=k_com.at[r],
                send_sem=ksend.at[j], recv_sem=krecv.at[r],
                device_id=(mx, my, p), device_id_type=pl.DeviceIdType.MESH,
            )
            rv = pltpu.make_async_remote_copy(
                src_ref=vloc, dst_ref=v_com.at[r],
                send_sem=vsend.at[j], recv_sem=vrecv.at[r],
                device_id=(mx, my, p), device_id_type=pl.DeviceIdType.MESH,
            )
            rk.start()
            rv.start()
            sends.append((rk, rv))

        def chunk_loop(k_at, v_at, first, last):
            def bh_step(bh, _):
                b = bh // H
                hh = bh % H
                q = qt[b, hh]
                kT = k_at(b, hh)
                vT = v_at(b, hh)
                s = lax.dot_general(
                    q, kT, (((1,), (0,)), ((), ())),
                    preferred_element_type=jnp.float32,
                )
                p = jnp.exp(s)
                pv = lax.dot_general(
                    p.astype(jnp.bfloat16), vT, (((1,), (1,)), ((), ())),
                    preferred_element_type=jnp.float32,
                )
                lv = jnp.sum(p, axis=1, keepdims=True)
                if first:
                    a = pv
                    l = lv
                else:
                    a = acc[b, hh] + pv
                    l = lsum[b, hh] + lv
                if last:
                    out_ref[b, :, hh, :] = a / l
                else:
                    acc[b, hh] = a
                    lsum[b, hh] = l
                return 0

            lax.fori_loop(0, B * H, bh_step, 0)

        if _ABLATE == "compute":
            chunk_loop(lambda b, hh: kloc[b, hh], lambda b, hh: vloc[b, hh],
                       first=True, last=False)
            for r in range(N_DEV - 1):
                chunk_loop(lambda b, hh: kloc[b, hh],
                           lambda b, hh: vloc[b, hh],
                           first=False, last=(r == N_DEV - 2))
        else:
            if _ABLATE != "comm":
                chunk_loop(lambda b, hh: kloc[b, hh],
                           lambda b, hh: vloc[b, hh],
                           first=True, last=False)

            for r in range(N_DEV - 1):
                wk = pltpu.make_async_remote_copy(
                    src_ref=kloc, dst_ref=k_com.at[r],
                    send_sem=ksend.at[0], recv_sem=krecv.at[r],
                    device_id=(mx, my, mz),
                    device_id_type=pl.DeviceIdType.MESH,
                )
                wv = pltpu.make_async_remote_copy(
                    src_ref=vloc, dst_ref=v_com.at[r],
                    send_sem=vsend.at[0], recv_sem=vrecv.at[r],
                    device_id=(mx, my, mz),
                    device_id_type=pl.DeviceIdType.MESH,
                )
                wk.wait_recv()
                wv.wait_recv()
                if _ABLATE != "comm":
                    chunk_loop(lambda b, hh, r=r: k_com[r, b, hh],
                               lambda b, hh, r=r: v_com[r, b, hh],
                               first=False, last=(r == N_DEV - 2))
            if _ABLATE == "comm":
                out_ref[...] = q_ref[...]

        for rk, rv in sends:
            rk.wait_send()
            rv.wait_send()

    return pl.pallas_call(
        body,
        out_shape=jax.ShapeDtypeStruct((B, S, H, D), jnp.float32),
        in_specs=[pl.BlockSpec(memory_space=pltpu.VMEM)] * 3,
        out_specs=pl.BlockSpec(memory_space=pltpu.VMEM),
        scratch_shapes=[
            pltpu.VMEM((B, H, S, D), jnp.bfloat16),
            pltpu.VMEM((B, H, D, S), jnp.bfloat16),
            pltpu.VMEM((B, H, D, S), jnp.bfloat16),
            pltpu.VMEM((N_DEV - 1, B, H, D, S), jnp.bfloat16),
            pltpu.VMEM((N_DEV - 1, B, H, D, S), jnp.bfloat16),
            pltpu.VMEM((B, H, S, D), jnp.float32),
            pltpu.VMEM((B, H, S, 1), jnp.float32),
            pltpu.SemaphoreType.DMA((N_DEV - 1,)),
            pltpu.SemaphoreType.DMA((N_DEV - 1,)),
            pltpu.SemaphoreType.DMA((N_DEV - 1,)),
            pltpu.SemaphoreType.DMA((N_DEV - 1,)),
        ],
        compiler_params=pltpu.CompilerParams(
            collective_id=0,
            vmem_limit_bytes=100 * 1024 * 1024,
        ),
    )(Q, K, V)
